# baseline (speedup 1.0000x reference)
import jax
import jax.numpy as jnp
from jax import lax
from jax.experimental import pallas as pl
from jax.experimental.pallas import tpu as pltpu
from jax.experimental.pallas import tpu_sc as plsc

N, D = 10000, 128
LAMB = 0.7
CH = 80
NCHUNK = N // CH
NW = 32
NITER = (NCHUNK + NW - 1) // NW


def _mix_body(x_hbm, idx_hbm, out_hbm,
              idx_v, xa0, xa1, xb0, xb1, spm,
              isem, dsem0, dsem1, t1sem0, t1sem1, t2sem0, t2sem1):
    info = plsc.get_sparse_core_info()
    sid = lax.axis_index("s")
    wid = sid * info.num_cores + lax.axis_index("c")

    xa = (xa0, xa1)
    xb = (xb0, xb1)
    dsem = (dsem0, dsem1)
    t1sem = (t1sem0, t1sem1)
    t2sem = (t2sem0, t2sem1)

    last = NCHUNK - 1
    base = [None] * NITER
    icopy = [None] * NITER
    for i in range(NITER):
        c = jnp.minimum(wid + i * NW, last)
        base[i] = c * CH
        icopy[i] = pltpu.async_copy(
            idx_hbm.at[pl.ds(base[i], CH)], idx_v.at[i], isem)

    gcopy = [None] * NITER
    lcopy = [None] * NITER
    s1copy = [None] * NITER
    s2copy = [None] * NITER

    def launch(i):
        b = i % 2
        icopy[i].wait()
        gcopy[i] = pltpu.async_copy(x_hbm.at[idx_v.at[i]], xb[b], dsem[b])
        lcopy[i] = pltpu.async_copy(x_hbm.at[pl.ds(base[i], CH)], xa[b],
                                    dsem[b])

    launch(0)
    for i in range(NITER):
        b = i % 2
        if i + 1 < NITER:
            if i - 1 >= 0:
                s1copy[i - 1].wait()
                s2copy[i - 1] = pltpu.async_copy(
                    spm.at[sid, (i - 1) % 2],
                    out_hbm.at[pl.ds(base[i - 1], CH)], t2sem[(i - 1) % 2])
            launch(i + 1)
        gcopy[i].wait()
        lcopy[i].wait()

        def row_body(r, rcarry):
            for cc in range(D // 16):
                s = pl.ds(cc * 16, 16)
                xa[b][r, s] = LAMB * xa[b][r, s] + (1.0 - LAMB) * xb[b][r, s]
            return rcarry

        lax.fori_loop(0, CH, row_body, 0, unroll=False)
        if i - 2 >= 0:
            s2copy[i - 2].wait()
        s1copy[i] = pltpu.async_copy(xa[b], spm.at[sid, b], t1sem[b])

    for i in (NITER - 2, NITER - 1):
        s1copy[i].wait()
        s2copy[i] = pltpu.async_copy(
            spm.at[sid, i % 2], out_hbm.at[pl.ds(base[i], CH)], t2sem[i % 2])
    s2copy[NITER - 2].wait()
    s2copy[NITER - 1].wait()


@jax.jit
def _mix(x, idx32):
    mesh = plsc.VectorSubcoreMesh(core_axis_name="c", subcore_axis_name="s")
    f = pl.kernel(
        _mix_body,
        mesh=mesh,
        out_type=jax.ShapeDtypeStruct((N, D), jnp.float32),
        scratch_types=[
            pltpu.VMEM((NITER, CH), jnp.int32),
            pltpu.VMEM((CH, D), jnp.float32),
            pltpu.VMEM((CH, D), jnp.float32),
            pltpu.VMEM((CH, D), jnp.float32),
            pltpu.VMEM((CH, D), jnp.float32),
            pltpu.VMEM_SHARED((16, 2, CH, D), jnp.float32),
            pltpu.SemaphoreType.DMA,
            pltpu.SemaphoreType.DMA,
            pltpu.SemaphoreType.DMA,
            pltpu.SemaphoreType.DMA,
            pltpu.SemaphoreType.DMA,
            pltpu.SemaphoreType.DMA,
            pltpu.SemaphoreType.DMA,
        ],
    )
    return f(x, idx32)


def kernel(x, y, edge_index, train_mask, test_mask, pair_idx):
    x_mix = _mix(x, pair_idx.astype(jnp.int32))
    new_y = y.astype(jnp.int32)
    return (x_mix, new_y, edge_index, train_mask, test_mask)


# speedup vs baseline: 1.0086x; 1.0086x over previous
import jax
import jax.numpy as jnp
from jax import lax
from jax.experimental import pallas as pl
from jax.experimental.pallas import tpu as pltpu
from jax.experimental.pallas import tpu_sc as plsc

N, D = 10000, 128
LAMB = 0.7
CH = 80
NCHUNK = N // CH
NW = 32
NITER = (NCHUNK + NW - 1) // NW


def _mix_body(x_hbm, idx_hbm, out_hbm,
              idx_v, xa0, xa1, xb0, xb1,
              isem, dsem0, dsem1, ssem0, ssem1):
    info = plsc.get_sparse_core_info()
    wid = lax.axis_index("s") * info.num_cores + lax.axis_index("c")

    xa = (xa0, xa1)
    xb = (xb0, xb1)
    dsem = (dsem0, dsem1)
    ssem = (ssem0, ssem1)

    last = NCHUNK - 1
    base = [None] * NITER
    icopy = [None] * NITER
    for i in range(NITER):
        c = jnp.minimum(wid + i * NW, last)
        base[i] = c * CH
        icopy[i] = pltpu.async_copy(
            idx_hbm.at[pl.ds(base[i], CH)], idx_v.at[i], isem)

    gcopy = [None] * NITER
    lcopy = [None] * NITER
    scopy = [None] * NITER

    def launch(i):
        b = i % 2
        icopy[i].wait()
        gcopy[i] = pltpu.async_copy(x_hbm.at[idx_v.at[i]], xb[b], dsem[b])
        lcopy[i] = pltpu.async_copy(x_hbm.at[pl.ds(base[i], CH)], xa[b],
                                    dsem[b])

    launch(0)
    for i in range(NITER):
        b = i % 2
        if i + 1 < NITER:
            if i - 1 >= 0:
                scopy[i - 1].wait()
            launch(i + 1)
        gcopy[i].wait()
        lcopy[i].wait()

        def row_body(r, rcarry):
            for cc in range(D // 16):
                s = pl.ds(cc * 16, 16)
                xa[b][r, s] = LAMB * xa[b][r, s] + (1.0 - LAMB) * xb[b][r, s]
            return rcarry

        lax.fori_loop(0, CH, row_body, 0, unroll=False)
        scopy[i] = pltpu.async_copy(xa[b], out_hbm.at[pl.ds(base[i], CH)],
                                    ssem[b])

    scopy[NITER - 2].wait()
    scopy[NITER - 1].wait()


@jax.jit
def _mix(x, idx32):
    mesh = plsc.VectorSubcoreMesh(core_axis_name="c", subcore_axis_name="s")
    f = pl.kernel(
        _mix_body,
        mesh=mesh,
        out_type=jax.ShapeDtypeStruct((N, D), jnp.float32),
        scratch_types=[
            pltpu.VMEM((NITER, CH), jnp.int32),
            pltpu.VMEM((CH, D), jnp.float32),
            pltpu.VMEM((CH, D), jnp.float32),
            pltpu.VMEM((CH, D), jnp.float32),
            pltpu.VMEM((CH, D), jnp.float32),
            pltpu.SemaphoreType.DMA,
            pltpu.SemaphoreType.DMA,
            pltpu.SemaphoreType.DMA,
            pltpu.SemaphoreType.DMA,
            pltpu.SemaphoreType.DMA,
        ],
    )
    return f(x, idx32)


def kernel(x, y, edge_index, train_mask, test_mask, pair_idx):
    x_mix = _mix(x, pair_idx.astype(jnp.int32))
    new_y = y.astype(jnp.int32)
    return (x_mix, new_y, edge_index, train_mask, test_mask)
